# baseline (speedup 1.0000x reference)
import functools

import jax
import jax.numpy as jnp
from jax.experimental import pallas as pl
from jax.experimental.pallas import tpu as pltpu

B, LQ, C, N = 32, 16, 256, 64
NUM_REL = 300
LR = 10
ENT_DIM = 100
WORD_DIM = 300
H = 64
HL = 32
CB = 64


def _lrelu(x):
    return jnp.where(x >= 0, x, 0.01 * x)


def _dot_t(a, b):
    return jax.lax.dot_general(a, b, (((1,), (1,)), ((), ())),
                               preferred_element_type=jnp.float32)


def _encode_kernel(xq_ref, xr_ref, qm_ref, rm_ref,
                   wih_f_ref, whh_f_ref, b_f_ref,
                   wih_b_ref, whh_b_ref, b_b_ref,
                   attn_r_ref, attn_q_ref,
                   q_emb_ref, q_vec_ref, rel_enc_ref,
                   relh_ref):
    wih_f = wih_f_ref[...]
    whh_f = whh_f_ref[...]
    b_f = b_f_ref[...]
    wih_b = wih_b_ref[...]
    whh_b = whh_b_ref[...]
    b_b = b_b_ref[...]

    def lstm_step(x_proj, m_col, h, c, whh):
        gates = x_proj + _dot_t(h, whh)
        i = jax.nn.sigmoid(gates[:, :HL])
        f = jax.nn.sigmoid(gates[:, HL:2 * HL])
        g = jnp.tanh(gates[:, 2 * HL:3 * HL])
        o = jax.nn.sigmoid(gates[:, 3 * HL:])
        c_new = f * c + i * g
        h_new = o * jnp.tanh(c_new)
        h2 = m_col * h_new + (1.0 - m_col) * h
        c2 = m_col * c_new + (1.0 - m_col) * c
        return h2, c2, h_new * m_col

    xq = xq_ref[...]
    proj_f = (_dot_t(xq, wih_f) + b_f).reshape(B, LQ, 4 * HL)
    proj_b = (_dot_t(xq, wih_b) + b_b).reshape(B, LQ, 4 * HL)
    h = jnp.zeros((B, HL), jnp.float32)
    c = jnp.zeros((B, HL), jnp.float32)
    outs_f = [None] * LQ
    for t in range(LQ):
        m_col = qm_ref[:, t:t + 1]
        h, c, o = lstm_step(proj_f[:, t, :], m_col, h, c, whh_f_ref[...])
        outs_f[t] = o
    h = jnp.zeros((B, HL), jnp.float32)
    c = jnp.zeros((B, HL), jnp.float32)
    outs_b = [None] * LQ
    for s in range(LQ):
        t = LQ - 1 - s
        m_col = qm_ref[:, t:t + 1]
        h, c, o = lstm_step(proj_b[:, t, :], m_col, h, c, whh_b_ref[...])
        outs_b[t] = o
    for t in range(LQ):
        q_emb_ref[:, t, :] = jnp.concatenate([outs_f[t], outs_b[t]], axis=1)

    attn_q = attn_q_ref[...]
    mx = jnp.full((B, 1), -jnp.inf)
    scores = [None] * LQ
    for t in range(LQ):
        sc = jnp.sum(q_emb_ref[:, t, :] * attn_q, axis=1, keepdims=True)
        sc = sc - (1.0 - qm_ref[:, t:t + 1]) * 1e8
        scores[t] = sc
        mx = jnp.maximum(mx, sc)
    ssum = jnp.zeros((B, 1), jnp.float32)
    qv = jnp.zeros((B, H), jnp.float32)
    for t in range(LQ):
        e = jnp.exp(scores[t] - mx)
        ssum = ssum + e
        qv = qv + e * q_emb_ref[:, t, :]
    q_vec_ref[:, 0, :] = qv / ssum

    xr = xr_ref[...]
    rproj_f = (_dot_t(xr, wih_f) + b_f).reshape(NUM_REL, LR, 4 * HL)
    rproj_b = (_dot_t(xr, wih_b) + b_b).reshape(NUM_REL, LR, 4 * HL)
    h = jnp.zeros((NUM_REL, HL), jnp.float32)
    c = jnp.zeros((NUM_REL, HL), jnp.float32)
    routs_f = [None] * LR
    for t in range(LR):
        m_col = rm_ref[:, t:t + 1]
        h, c, o = lstm_step(rproj_f[:, t, :], m_col, h, c, whh_f_ref[...])
        routs_f[t] = o
    h = jnp.zeros((NUM_REL, HL), jnp.float32)
    c = jnp.zeros((NUM_REL, HL), jnp.float32)
    for s in range(LR):
        t = LR - 1 - s
        m_col = rm_ref[:, t:t + 1]
        h, c, o = lstm_step(rproj_b[:, t, :], m_col, h, c, whh_b_ref[...])
        relh_ref[t] = jnp.concatenate([routs_f[t], o], axis=1)

    attn_r = attn_r_ref[...]
    mx = jnp.full((NUM_REL, 1), -jnp.inf)
    rscores = [None] * LR
    for t in range(LR):
        sc = jnp.sum(relh_ref[t] * attn_r, axis=1, keepdims=True)
        sc = sc - (1.0 - rm_ref[:, t:t + 1]) * 1e8
        rscores[t] = sc
        mx = jnp.maximum(mx, sc)
    ssum = jnp.zeros((NUM_REL, 1), jnp.float32)
    rv = jnp.zeros((NUM_REL, H), jnp.float32)
    for t in range(LR):
        e = jnp.exp(rscores[t] - mx)
        ssum = ssum + e
        rv = rv + e * relh_ref[t]
    rel_enc_ref[...] = rv / ssum


def _main_kernel(q_emb_ref, qids_ref, rel_ref, rid_ref, eid_ref, qe_ref,
                 ent_ref,
                 ent_lin_W_ref, ent_lin_b_ref,
                 comb_qrel_W_ref, comb_qrel_b_ref,
                 comb_q_W_ref, comb_q_b_ref,
                 kg_prop_W_ref, kg_prop_b_ref,
                 kg_gate_W_ref, kg_gate_b_ref,
                 q_vec_ref, out_ref):
    rel = rel_ref[...]
    q_emb = q_emb_ref[0]

    qrel = _dot_t(q_emb, rel)
    mq = jnp.max(qrel, axis=1, keepdims=True)
    E = jnp.exp(qrel - mq)
    qrelT = _dot_t(rel, q_emb)
    mqT = jnp.max(qrelT, axis=0, keepdims=True)
    ET = jnp.exp(qrelT - mqT)

    mask_row = (qids_ref[0] != 1).astype(jnp.float32)
    qm = qrelT - (1.0 - mask_row) * 1e20
    qm = qm - jnp.max(qm, axis=1, keepdims=True)
    SqT = jnp.exp(qm)
    SqT = SqT / jnp.sum(SqT, axis=1, keepdims=True)
    Rq = jnp.dot(SqT, q_emb, preferred_element_type=jnp.float32)
    s300 = jnp.sum(rel * Rq, axis=1, keepdims=True)

    Wqr1 = comb_qrel_W_ref[...][:, :H]
    Wqr2 = comb_qrel_W_ref[...][:, H:]
    Aq = _dot_t(q_emb, Wqr1) + comb_qrel_b_ref[...]

    el = _lrelu(_dot_t(ent_ref[0], ent_lin_W_ref[...]) + ent_lin_b_ref[...])

    qe_col = qe_ref[0]

    iota_rel = jax.lax.broadcasted_iota(jnp.int32, (1, 1, NUM_REL), 2)
    iota_ent = jax.lax.broadcasted_iota(jnp.int32, (1, 1, C), 2)

    ent_new_blks = []
    mg_blks = []
    rel_agg_blks = []
    z_blks = []
    for cb in range(C // CB):
        c0 = cb * CB
        rid3 = rid_ref[0, c0:c0 + CB, :][:, :, None]
        eid3 = eid_ref[0, c0:c0 + CB, :][:, :, None]
        oh_rel = (rid3 == iota_rel).astype(jnp.float32)
        oh_ent = (eid3 == iota_ent).astype(jnp.float32)
        cnt = jnp.sum(oh_rel, axis=1)

        oh_rel2 = oh_rel.reshape(CB * N, NUM_REL)
        oh_ent2 = oh_ent.reshape(CB * N, C)
        s_n = jnp.dot(oh_rel2, s300, preferred_element_type=jnp.float32)
        seed = jnp.dot(oh_ent2, qe_col, preferred_element_type=jnp.float32)
        sv = (s_n * seed).reshape(CB, N, 1)
        g = jnp.exp(sv - jnp.max(sv, axis=1, keepdims=True))
        z_blks.append(jnp.sum(g, axis=1))
        cntg = jnp.sum(oh_rel * g, axis=1)
        mg_blks.append(jnp.sum(oh_ent * g, axis=1))
        rel_agg_blks.append(jnp.dot(cntg, rel, preferred_element_type=jnp.float32))

        den = jnp.dot(cnt, ET, preferred_element_type=jnp.float32)
        qn = jnp.full((CB, H), -1e30)
        for q in range(LQ):
            w = cnt * E[q:q + 1, :]
            nq = jnp.dot(w, rel, preferred_element_type=jnp.float32)
            rq = nq / den[:, q:q + 1]
            qn = jnp.maximum(qn, jnp.tanh(_dot_t(rq, Wqr2) + Aq[q:q + 1, :]))
        cqW = comb_q_W_ref[...]
        ent_new_blks.append(_lrelu(_dot_t(el[c0:c0 + CB], cqW[:, :H])
                                   + _dot_t(qn, cqW[:, H:])
                                   + comb_q_b_ref[...]))

    ent_new = jnp.concatenate(ent_new_blks, axis=0)
    mg = jnp.concatenate(mg_blks, axis=0)
    rel_agg = jnp.concatenate(rel_agg_blks, axis=0)
    zz = jnp.concatenate(z_blks, axis=0)

    ent_agg = jnp.dot(mg, ent_new, preferred_element_type=jnp.float32)
    pW = kg_prop_W_ref[...]
    agg = (_dot_t(rel_agg, pW[:, :H]) + _dot_t(ent_agg, pW[:, H:])) / zz \
        + kg_prop_b_ref[...]
    gW = kg_gate_W_ref[...]
    gate = jax.nn.sigmoid(_dot_t(agg, gW[:, :H]) + _dot_t(ent_new, gW[:, H:])
                          + kg_gate_b_ref[...])
    ent2 = gate * _lrelu(agg) + (1.0 - gate) * ent_new
    out_ref[0] = _dot_t(q_vec_ref[0], ent2)


@jax.jit
def kernel(questions, candidate_entities, entity_link_ents, entity_link_rels,
           rel_word_ids, query_entities, entity_table, word_table,
           ent_lin_W, ent_lin_b,
           lstm_Wih_f, lstm_Whh_f, lstm_bih_f, lstm_bhh_f,
           lstm_Wih_b, lstm_Whh_b, lstm_bih_b, lstm_bhh_b,
           attn_r_w, attn_q_w, comb_qrel_W, comb_qrel_b,
           comb_q_W, comb_q_b, kg_prop_W, kg_prop_b, kg_gate_W, kg_gate_b):
    f32 = jnp.float32

    qids = questions.astype(jnp.int32)
    rids = rel_word_ids.astype(jnp.int32)
    xq = word_table[qids].reshape(B * LQ, WORD_DIM)
    xr = word_table[rids].reshape(NUM_REL * LR, WORD_DIM)
    ent_rows = entity_table[candidate_entities]

    qm = (qids != 1).astype(f32)
    rm = (rids != 1).astype(f32)

    b_f = (lstm_bih_f + lstm_bhh_f).reshape(1, 4 * HL)
    b_b = (lstm_bih_b + lstm_bhh_b).reshape(1, 4 * HL)

    q_emb, q_vec, rel_encoded = pl.pallas_call(
        _encode_kernel,
        out_shape=[
            jax.ShapeDtypeStruct((B, LQ, H), f32),
            jax.ShapeDtypeStruct((B, 1, H), f32),
            jax.ShapeDtypeStruct((NUM_REL, H), f32),
        ],
        scratch_shapes=[pltpu.VMEM((LR, NUM_REL, H), f32)],
    )(xq, xr, qm, rm,
      lstm_Wih_f, lstm_Whh_f, b_f,
      lstm_Wih_b, lstm_Whh_b, b_b,
      attn_r_w.reshape(1, H), attn_q_w.reshape(1, H))

    qids3 = questions.astype(jnp.int32).reshape(B, 1, LQ)
    qe3 = query_entities.astype(f32).reshape(B, C, 1)
    rid = entity_link_rels.astype(jnp.int32)
    eid = entity_link_ents.astype(jnp.int32)

    full = lambda shape: pl.BlockSpec(shape, lambda b: tuple(0 for _ in shape))
    row = lambda shape: pl.BlockSpec(shape, lambda b: (b,) + tuple(0 for _ in shape[1:]))

    out = pl.pallas_call(
        _main_kernel,
        grid=(B,),
        in_specs=[
            row((1, LQ, H)),
            row((1, 1, LQ)),
            full((NUM_REL, H)),
            row((1, C, N)),
            row((1, C, N)),
            row((1, C, 1)),
            row((1, C, ENT_DIM)),
            full((H, ENT_DIM)),
            full((1, H)),
            full((H, 2 * H)),
            full((1, H)),
            full((H, 2 * H)),
            full((1, H)),
            full((H, 2 * H)),
            full((1, H)),
            full((H, 2 * H)),
            full((1, H)),
            row((1, 1, H)),
        ],
        out_specs=pl.BlockSpec((1, 1, C), lambda b: (b, 0, 0)),
        out_shape=jax.ShapeDtypeStruct((B, 1, C), f32),
        compiler_params=pltpu.CompilerParams(
            dimension_semantics=("arbitrary",),
        ),
    )(q_emb, qids3, rel_encoded, rid, eid, qe3, ent_rows,
      ent_lin_W, ent_lin_b.reshape(1, H),
      comb_qrel_W, comb_qrel_b.reshape(1, H),
      comb_q_W, comb_q_b.reshape(1, H),
      kg_prop_W, kg_prop_b.reshape(1, H),
      kg_gate_W, kg_gate_b.reshape(1, H),
      q_vec)
    return out.reshape(B, C)


# speedup vs baseline: 1.1862x; 1.1862x over previous
import functools

import jax
import jax.numpy as jnp
from jax.experimental import pallas as pl
from jax.experimental.pallas import tpu as pltpu
from jax.experimental.pallas import tpu_sc as plsc

B, LQ, C, N = 32, 16, 256, 64
NUM_REL = 300
LR = 10
ENT_DIM = 100
WORD_DIM = 300
H = 64
HL = 32
CB = 64

SC_NC, SC_NS = 2, 16
SC_NW = SC_NC * SC_NS
XQ_ROWS = B * LQ
XR_ROWS = 3072
ENT_ROWS = B * C
XQ_PW = XQ_ROWS // SC_NW
XR_PW = XR_ROWS // SC_NW
ENT_PW = ENT_ROWS // SC_NW
WPROJ = 8 * HL
EPROJ = 128


def _gather_sc_kernel(wt_ref, et_ref, qidx_ref, ridx_ref, eidx_ref,
                      xq_out, xr_out, ent_out,
                      qi_v, ri_v, ei0_v, ei1_v, xq_v, xr_v, ent_v,
                      s0, s1, s2, s3):
    wid = jax.lax.axis_index("s") * SC_NC + jax.lax.axis_index("c")
    qb = wid * XQ_PW
    rb = wid * XR_PW
    eb = wid * ENT_PW
    pltpu.sync_copy(qidx_ref.at[pl.ds(qb, XQ_PW)], qi_v)
    pltpu.sync_copy(ridx_ref.at[pl.ds(rb, XR_PW)], ri_v)
    pltpu.sync_copy(eidx_ref.at[pl.ds(eb, 128)], ei0_v)
    pltpu.sync_copy(eidx_ref.at[pl.ds(eb + 128, 128)], ei1_v)
    c0 = pltpu.async_copy(wt_ref.at[qi_v], xq_v, s0)
    c1 = pltpu.async_copy(wt_ref.at[ri_v], xr_v, s1)
    c2 = pltpu.async_copy(et_ref.at[ei0_v], ent_v.at[pl.ds(0, 128)], s2)
    c3 = pltpu.async_copy(et_ref.at[ei1_v], ent_v.at[pl.ds(128, 128)], s3)
    c0.wait()
    c1.wait()
    c2.wait()
    c3.wait()
    pltpu.sync_copy(xq_v, xq_out.at[pl.ds(qb, XQ_PW)])
    pltpu.sync_copy(xr_v, xr_out.at[pl.ds(rb, XR_PW)])
    pltpu.sync_copy(ent_v, ent_out.at[pl.ds(eb, ENT_PW)])


_gather_sc = functools.partial(
    pl.kernel,
    mesh=plsc.VectorSubcoreMesh(core_axis_name="c", subcore_axis_name="s"),
    out_type=[
        jax.ShapeDtypeStruct((XQ_ROWS, WPROJ), jnp.float32),
        jax.ShapeDtypeStruct((XR_ROWS, WPROJ), jnp.float32),
        jax.ShapeDtypeStruct((ENT_ROWS, EPROJ), jnp.float32),
    ],
    scratch_types=[
        pltpu.VMEM((XQ_PW,), jnp.int32),
        pltpu.VMEM((XR_PW,), jnp.int32),
        pltpu.VMEM((128,), jnp.int32),
        pltpu.VMEM((128,), jnp.int32),
        pltpu.VMEM((XQ_PW, WPROJ), jnp.float32),
        pltpu.VMEM((XR_PW, WPROJ), jnp.float32),
        pltpu.VMEM((ENT_PW, EPROJ), jnp.float32),
        pltpu.SemaphoreType.DMA,
        pltpu.SemaphoreType.DMA,
        pltpu.SemaphoreType.DMA,
        pltpu.SemaphoreType.DMA,
    ],
)(_gather_sc_kernel)


def _proj_words_kernel(wt_ref, wcat_ref, out_ref):
    out_ref[...] = jnp.dot(wt_ref[...], wcat_ref[...],
                           preferred_element_type=jnp.float32)


def _proj_ents_kernel(et_ref, ew_ref, out_ref):
    out_ref[...] = _dot_t(et_ref[...], ew_ref[...])


def _lrelu(x):
    return jnp.where(x >= 0, x, 0.01 * x)


def _dot_t(a, b):
    return jax.lax.dot_general(a, b, (((1,), (1,)), ((), ())),
                               preferred_element_type=jnp.float32)


def _encode_kernel(xq_ref, xr_ref, qm_ref, rm_ref,
                   whh_f_ref, b_f_ref,
                   whh_b_ref, b_b_ref,
                   attn_r_ref, attn_q_ref,
                   q_emb_ref, q_vec_ref, rel_enc_ref,
                   relh_ref):
    b_f = b_f_ref[...]
    b_b = b_b_ref[...]

    def lstm_step(x_proj, m_col, h, c, whh):
        gates = x_proj + _dot_t(h, whh)
        i = jax.nn.sigmoid(gates[:, :HL])
        f = jax.nn.sigmoid(gates[:, HL:2 * HL])
        g = jnp.tanh(gates[:, 2 * HL:3 * HL])
        o = jax.nn.sigmoid(gates[:, 3 * HL:])
        c_new = f * c + i * g
        h_new = o * jnp.tanh(c_new)
        h2 = m_col * h_new + (1.0 - m_col) * h
        c2 = m_col * c_new + (1.0 - m_col) * c
        return h2, c2, h_new * m_col

    xq = xq_ref[...]
    proj_f = (xq[:, :4 * HL] + b_f).reshape(B, LQ, 4 * HL)
    proj_b = (xq[:, 4 * HL:] + b_b).reshape(B, LQ, 4 * HL)
    h = jnp.zeros((B, HL), jnp.float32)
    c = jnp.zeros((B, HL), jnp.float32)
    outs_f = [None] * LQ
    for t in range(LQ):
        m_col = qm_ref[:, t:t + 1]
        h, c, o = lstm_step(proj_f[:, t, :], m_col, h, c, whh_f_ref[...])
        outs_f[t] = o
    h = jnp.zeros((B, HL), jnp.float32)
    c = jnp.zeros((B, HL), jnp.float32)
    outs_b = [None] * LQ
    for s in range(LQ):
        t = LQ - 1 - s
        m_col = qm_ref[:, t:t + 1]
        h, c, o = lstm_step(proj_b[:, t, :], m_col, h, c, whh_b_ref[...])
        outs_b[t] = o
    for t in range(LQ):
        q_emb_ref[:, t, :] = jnp.concatenate([outs_f[t], outs_b[t]], axis=1)

    attn_q = attn_q_ref[...]
    mx = jnp.full((B, 1), -jnp.inf)
    scores = [None] * LQ
    for t in range(LQ):
        sc = jnp.sum(q_emb_ref[:, t, :] * attn_q, axis=1, keepdims=True)
        sc = sc - (1.0 - qm_ref[:, t:t + 1]) * 1e8
        scores[t] = sc
        mx = jnp.maximum(mx, sc)
    ssum = jnp.zeros((B, 1), jnp.float32)
    qv = jnp.zeros((B, H), jnp.float32)
    for t in range(LQ):
        e = jnp.exp(scores[t] - mx)
        ssum = ssum + e
        qv = qv + e * q_emb_ref[:, t, :]
    q_vec_ref[:, 0, :] = qv / ssum

    xr = xr_ref[...][:NUM_REL * LR]
    rproj_f = (xr[:, :4 * HL] + b_f).reshape(NUM_REL, LR, 4 * HL)
    rproj_b = (xr[:, 4 * HL:] + b_b).reshape(NUM_REL, LR, 4 * HL)
    h = jnp.zeros((NUM_REL, HL), jnp.float32)
    c = jnp.zeros((NUM_REL, HL), jnp.float32)
    routs_f = [None] * LR
    for t in range(LR):
        m_col = rm_ref[:, t:t + 1]
        h, c, o = lstm_step(rproj_f[:, t, :], m_col, h, c, whh_f_ref[...])
        routs_f[t] = o
    h = jnp.zeros((NUM_REL, HL), jnp.float32)
    c = jnp.zeros((NUM_REL, HL), jnp.float32)
    for s in range(LR):
        t = LR - 1 - s
        m_col = rm_ref[:, t:t + 1]
        h, c, o = lstm_step(rproj_b[:, t, :], m_col, h, c, whh_b_ref[...])
        relh_ref[t] = jnp.concatenate([routs_f[t], o], axis=1)

    attn_r = attn_r_ref[...]
    mx = jnp.full((NUM_REL, 1), -jnp.inf)
    rscores = [None] * LR
    for t in range(LR):
        sc = jnp.sum(relh_ref[t] * attn_r, axis=1, keepdims=True)
        sc = sc - (1.0 - rm_ref[:, t:t + 1]) * 1e8
        rscores[t] = sc
        mx = jnp.maximum(mx, sc)
    ssum = jnp.zeros((NUM_REL, 1), jnp.float32)
    rv = jnp.zeros((NUM_REL, H), jnp.float32)
    for t in range(LR):
        e = jnp.exp(rscores[t] - mx)
        ssum = ssum + e
        rv = rv + e * relh_ref[t]
    rel_enc_ref[...] = rv / ssum


def _main_kernel(q_emb_ref, qids_ref, rel_ref, rid_ref, eid_ref, qe_ref,
                 ent_ref,
                 ent_lin_b_ref,
                 comb_qrel_W_ref, comb_qrel_b_ref,
                 comb_q_W_ref, comb_q_b_ref,
                 kg_prop_W_ref, kg_prop_b_ref,
                 kg_gate_W_ref, kg_gate_b_ref,
                 q_vec_ref, out_ref):
    rel = rel_ref[...]
    q_emb = q_emb_ref[0]

    qrel = _dot_t(q_emb, rel)
    mq = jnp.max(qrel, axis=1, keepdims=True)
    E = jnp.exp(qrel - mq)
    qrelT = _dot_t(rel, q_emb)
    mqT = jnp.max(qrelT, axis=0, keepdims=True)
    ET = jnp.exp(qrelT - mqT)

    mask_row = (qids_ref[0] != 1).astype(jnp.float32)
    qm = qrelT - (1.0 - mask_row) * 1e20
    qm = qm - jnp.max(qm, axis=1, keepdims=True)
    SqT = jnp.exp(qm)
    SqT = SqT / jnp.sum(SqT, axis=1, keepdims=True)
    Rq = jnp.dot(SqT, q_emb, preferred_element_type=jnp.float32)
    s300 = jnp.sum(rel * Rq, axis=1, keepdims=True)

    Wqr1 = comb_qrel_W_ref[...][:, :H]
    Wqr2 = comb_qrel_W_ref[...][:, H:]
    Aq = _dot_t(q_emb, Wqr1) + comb_qrel_b_ref[...]

    el = _lrelu(ent_ref[...][:, :H] + ent_lin_b_ref[...])

    qe_col = qe_ref[0]

    iota_rel = jax.lax.broadcasted_iota(jnp.int32, (1, 1, NUM_REL), 2)
    iota_ent = jax.lax.broadcasted_iota(jnp.int32, (1, 1, C), 2)

    ent_new_blks = []
    mg_blks = []
    rel_agg_blks = []
    z_blks = []
    for cb in range(C // CB):
        c0 = cb * CB
        rid3 = rid_ref[0, c0:c0 + CB, :][:, :, None]
        eid3 = eid_ref[0, c0:c0 + CB, :][:, :, None]
        oh_rel = (rid3 == iota_rel).astype(jnp.float32)
        oh_ent = (eid3 == iota_ent).astype(jnp.float32)
        cnt = jnp.sum(oh_rel, axis=1)

        oh_rel2 = oh_rel.reshape(CB * N, NUM_REL)
        oh_ent2 = oh_ent.reshape(CB * N, C)
        s_n = jnp.dot(oh_rel2, s300, preferred_element_type=jnp.float32)
        seed = jnp.dot(oh_ent2, qe_col, preferred_element_type=jnp.float32)
        sv = (s_n * seed).reshape(CB, N, 1)
        g = jnp.exp(sv - jnp.max(sv, axis=1, keepdims=True))
        z_blks.append(jnp.sum(g, axis=1))
        cntg = jnp.sum(oh_rel * g, axis=1)
        mg_blks.append(jnp.sum(oh_ent * g, axis=1))
        rel_agg_blks.append(jnp.dot(cntg, rel, preferred_element_type=jnp.float32))

        den = jnp.dot(cnt, ET, preferred_element_type=jnp.float32)
        qn = jnp.full((CB, H), -1e30)
        for q in range(LQ):
            w = cnt * E[q:q + 1, :]
            nq = jnp.dot(w, rel, preferred_element_type=jnp.float32)
            rq = nq / den[:, q:q + 1]
            qn = jnp.maximum(qn, jnp.tanh(_dot_t(rq, Wqr2) + Aq[q:q + 1, :]))
        cqW = comb_q_W_ref[...]
        ent_new_blks.append(_lrelu(_dot_t(el[c0:c0 + CB], cqW[:, :H])
                                   + _dot_t(qn, cqW[:, H:])
                                   + comb_q_b_ref[...]))

    ent_new = jnp.concatenate(ent_new_blks, axis=0)
    mg = jnp.concatenate(mg_blks, axis=0)
    rel_agg = jnp.concatenate(rel_agg_blks, axis=0)
    zz = jnp.concatenate(z_blks, axis=0)

    ent_agg = jnp.dot(mg, ent_new, preferred_element_type=jnp.float32)
    pW = kg_prop_W_ref[...]
    agg = (_dot_t(rel_agg, pW[:, :H]) + _dot_t(ent_agg, pW[:, H:])) / zz \
        + kg_prop_b_ref[...]
    gW = kg_gate_W_ref[...]
    gate = jax.nn.sigmoid(_dot_t(agg, gW[:, :H]) + _dot_t(ent_new, gW[:, H:])
                          + kg_gate_b_ref[...])
    ent2 = gate * _lrelu(agg) + (1.0 - gate) * ent_new
    out_ref[0] = _dot_t(q_vec_ref[0], ent2)


@jax.jit
def kernel(questions, candidate_entities, entity_link_ents, entity_link_rels,
           rel_word_ids, query_entities, entity_table, word_table,
           ent_lin_W, ent_lin_b,
           lstm_Wih_f, lstm_Whh_f, lstm_bih_f, lstm_bhh_f,
           lstm_Wih_b, lstm_Whh_b, lstm_bih_b, lstm_bhh_b,
           attn_r_w, attn_q_w, comb_qrel_W, comb_qrel_b,
           comb_q_W, comb_q_b, kg_prop_W, kg_prop_b, kg_gate_W, kg_gate_b):
    f32 = jnp.float32

    nw = word_table.shape[0]
    ne = entity_table.shape[0]
    wcat = jnp.concatenate([lstm_Wih_f.T, lstm_Wih_b.T], axis=1)
    ew_pad = jnp.zeros((EPROJ, ENT_DIM), f32).at[:H].set(ent_lin_W)

    wblk = 2000
    word_proj = pl.pallas_call(
        _proj_words_kernel,
        grid=(nw // wblk,),
        in_specs=[pl.BlockSpec((wblk, WORD_DIM), lambda i: (i, 0)),
                  pl.BlockSpec((WORD_DIM, WPROJ), lambda i: (0, 0))],
        out_specs=pl.BlockSpec((wblk, WPROJ), lambda i: (i, 0)),
        out_shape=jax.ShapeDtypeStruct((nw, WPROJ), f32),
    )(word_table, wcat)

    eblk = 2048
    ent_proj = pl.pallas_call(
        _proj_ents_kernel,
        grid=(pl.cdiv(ne, eblk),),
        in_specs=[pl.BlockSpec((eblk, ENT_DIM), lambda i: (i, 0)),
                  pl.BlockSpec((EPROJ, ENT_DIM), lambda i: (0, 0))],
        out_specs=pl.BlockSpec((eblk, EPROJ), lambda i: (i, 0)),
        out_shape=jax.ShapeDtypeStruct((ne, EPROJ), f32),
    )(entity_table, ew_pad)

    qids = questions.astype(jnp.int32)
    rids = rel_word_ids.astype(jnp.int32)
    qidx = qids.reshape(XQ_ROWS)
    ridx = jnp.concatenate(
        [rids.reshape(NUM_REL * LR),
         jnp.zeros((XR_ROWS - NUM_REL * LR,), jnp.int32)])
    eidx = candidate_entities.astype(jnp.int32).reshape(ENT_ROWS)
    xq, xr, ent_rows = _gather_sc(word_proj, ent_proj, qidx, ridx, eidx)

    qm = (qids != 1).astype(f32)
    rm = (rids != 1).astype(f32)

    b_f = (lstm_bih_f + lstm_bhh_f).reshape(1, 4 * HL)
    b_b = (lstm_bih_b + lstm_bhh_b).reshape(1, 4 * HL)

    q_emb, q_vec, rel_encoded = pl.pallas_call(
        _encode_kernel,
        out_shape=[
            jax.ShapeDtypeStruct((B, LQ, H), f32),
            jax.ShapeDtypeStruct((B, 1, H), f32),
            jax.ShapeDtypeStruct((NUM_REL, H), f32),
        ],
        scratch_shapes=[pltpu.VMEM((LR, NUM_REL, H), f32)],
    )(xq, xr, qm, rm,
      lstm_Whh_f, b_f,
      lstm_Whh_b, b_b,
      attn_r_w.reshape(1, H), attn_q_w.reshape(1, H))

    qids3 = questions.astype(jnp.int32).reshape(B, 1, LQ)
    qe3 = query_entities.astype(f32).reshape(B, C, 1)
    rid = entity_link_rels.astype(jnp.int32)
    eid = entity_link_ents.astype(jnp.int32)

    full = lambda shape: pl.BlockSpec(shape, lambda b: tuple(0 for _ in shape))
    row = lambda shape: pl.BlockSpec(shape, lambda b: (b,) + tuple(0 for _ in shape[1:]))

    out = pl.pallas_call(
        _main_kernel,
        grid=(B,),
        in_specs=[
            row((1, LQ, H)),
            row((1, 1, LQ)),
            full((NUM_REL, H)),
            row((1, C, N)),
            row((1, C, N)),
            row((1, C, 1)),
            pl.BlockSpec((C, EPROJ), lambda b: (b, 0)),
            full((1, H)),
            full((H, 2 * H)),
            full((1, H)),
            full((H, 2 * H)),
            full((1, H)),
            full((H, 2 * H)),
            full((1, H)),
            full((H, 2 * H)),
            full((1, H)),
            row((1, 1, H)),
        ],
        out_specs=pl.BlockSpec((1, 1, C), lambda b: (b, 0, 0)),
        out_shape=jax.ShapeDtypeStruct((B, 1, C), f32),
        compiler_params=pltpu.CompilerParams(
            dimension_semantics=("arbitrary",),
        ),
    )(q_emb, qids3, rel_encoded, rid, eid, qe3, ent_rows,
      ent_lin_b.reshape(1, H),
      comb_qrel_W, comb_qrel_b.reshape(1, H),
      comb_q_W, comb_q_b.reshape(1, H),
      kg_prop_W, kg_prop_b.reshape(1, H),
      kg_gate_W, kg_gate_b.reshape(1, H),
      q_vec)
    return out.reshape(B, C)


# speedup vs baseline: 1.2978x; 1.0940x over previous
import functools

import jax
import jax.numpy as jnp
from jax.experimental import pallas as pl
from jax.experimental.pallas import tpu as pltpu
from jax.experimental.pallas import tpu_sc as plsc

B, LQ, C, N = 32, 16, 256, 64
NUM_REL = 300
LR = 10
ENT_DIM = 100
WORD_DIM = 300
H = 64
HL = 32
CB = 64

SC_NC, SC_NS = 2, 16
SC_NW = SC_NC * SC_NS
XQ_ROWS = B * LQ
XR_ROWS = 3072
ENT_ROWS = B * C
XQ_PW = XQ_ROWS // SC_NW
XR_PW = XR_ROWS // SC_NW
ENT_PW = ENT_ROWS // SC_NW
WPROJ = 8 * HL
EPROJ = 128


def _gather_sc_kernel(wt_ref, et_ref, qidx_ref, ridx_ref, eidx_ref,
                      xq_out, xr_out, ent_out,
                      qi_v, ri_v, ei0_v, ei1_v, xq_v, xr_v, ent_v,
                      s0, s1, s2, s3):
    wid = jax.lax.axis_index("s") * SC_NC + jax.lax.axis_index("c")
    qb = wid * XQ_PW
    rb = wid * XR_PW
    eb = wid * ENT_PW
    pltpu.sync_copy(qidx_ref.at[pl.ds(qb, XQ_PW)], qi_v)
    pltpu.sync_copy(ridx_ref.at[pl.ds(rb, XR_PW)], ri_v)
    pltpu.sync_copy(eidx_ref.at[pl.ds(eb, 128)], ei0_v)
    pltpu.sync_copy(eidx_ref.at[pl.ds(eb + 128, 128)], ei1_v)
    c0 = pltpu.async_copy(wt_ref.at[qi_v], xq_v, s0)
    c1 = pltpu.async_copy(wt_ref.at[ri_v], xr_v, s1)
    c2 = pltpu.async_copy(et_ref.at[ei0_v], ent_v.at[pl.ds(0, 128)], s2)
    c3 = pltpu.async_copy(et_ref.at[ei1_v], ent_v.at[pl.ds(128, 128)], s3)
    c0.wait()
    c1.wait()
    c2.wait()
    c3.wait()
    pltpu.sync_copy(xq_v, xq_out.at[pl.ds(qb, XQ_PW)])
    pltpu.sync_copy(xr_v, xr_out.at[pl.ds(rb, XR_PW)])
    pltpu.sync_copy(ent_v, ent_out.at[pl.ds(eb, ENT_PW)])


_gather_sc = functools.partial(
    pl.kernel,
    mesh=plsc.VectorSubcoreMesh(core_axis_name="c", subcore_axis_name="s"),
    out_type=[
        jax.ShapeDtypeStruct((XQ_ROWS, WPROJ), jnp.float32),
        jax.ShapeDtypeStruct((XR_ROWS, WPROJ), jnp.float32),
        jax.ShapeDtypeStruct((ENT_ROWS, EPROJ), jnp.float32),
    ],
    scratch_types=[
        pltpu.VMEM((XQ_PW,), jnp.int32),
        pltpu.VMEM((XR_PW,), jnp.int32),
        pltpu.VMEM((128,), jnp.int32),
        pltpu.VMEM((128,), jnp.int32),
        pltpu.VMEM((XQ_PW, WPROJ), jnp.float32),
        pltpu.VMEM((XR_PW, WPROJ), jnp.float32),
        pltpu.VMEM((ENT_PW, EPROJ), jnp.float32),
        pltpu.SemaphoreType.DMA,
        pltpu.SemaphoreType.DMA,
        pltpu.SemaphoreType.DMA,
        pltpu.SemaphoreType.DMA,
    ],
)(_gather_sc_kernel)


def _proj_words_kernel(wt_ref, wcat_ref, out_ref):
    out_ref[...] = jnp.dot(wt_ref[...], wcat_ref[...],
                           preferred_element_type=jnp.float32)


def _proj_ents_kernel(et_ref, ew_ref, out_ref):
    out_ref[...] = _dot_t(et_ref[...], ew_ref[...])


def _lrelu(x):
    return jnp.where(x >= 0, x, 0.01 * x)


def _dot_t(a, b):
    return jax.lax.dot_general(a, b, (((1,), (1,)), ((), ())),
                               preferred_element_type=jnp.float32)


def _encode_kernel(xq_ref, xr_ref, qm_ref, rm_ref,
                   whh_f_ref, b_f_ref,
                   whh_b_ref, b_b_ref,
                   attn_r_ref, attn_q_ref,
                   q_emb_ref, q_vec_ref, rel_enc_ref,
                   relh_ref):
    b_f = b_f_ref[...]
    b_b = b_b_ref[...]

    def lstm_step(x_proj, m_col, h, c, whh):
        gates = x_proj + _dot_t(h, whh)
        i = jax.nn.sigmoid(gates[:, :HL])
        f = jax.nn.sigmoid(gates[:, HL:2 * HL])
        g = jnp.tanh(gates[:, 2 * HL:3 * HL])
        o = jax.nn.sigmoid(gates[:, 3 * HL:])
        c_new = f * c + i * g
        h_new = o * jnp.tanh(c_new)
        h2 = m_col * h_new + (1.0 - m_col) * h
        c2 = m_col * c_new + (1.0 - m_col) * c
        return h2, c2, h_new * m_col

    xq = xq_ref[...]
    proj_f = (xq[:, :4 * HL] + b_f).reshape(B, LQ, 4 * HL)
    proj_b = (xq[:, 4 * HL:] + b_b).reshape(B, LQ, 4 * HL)
    h = jnp.zeros((B, HL), jnp.float32)
    c = jnp.zeros((B, HL), jnp.float32)
    outs_f = [None] * LQ
    for t in range(LQ):
        m_col = qm_ref[:, t:t + 1]
        h, c, o = lstm_step(proj_f[:, t, :], m_col, h, c, whh_f_ref[...])
        outs_f[t] = o
    h = jnp.zeros((B, HL), jnp.float32)
    c = jnp.zeros((B, HL), jnp.float32)
    outs_b = [None] * LQ
    for s in range(LQ):
        t = LQ - 1 - s
        m_col = qm_ref[:, t:t + 1]
        h, c, o = lstm_step(proj_b[:, t, :], m_col, h, c, whh_b_ref[...])
        outs_b[t] = o
    for t in range(LQ):
        q_emb_ref[:, t, :] = jnp.concatenate([outs_f[t], outs_b[t]], axis=1)

    attn_q = attn_q_ref[...]
    mx = jnp.full((B, 1), -jnp.inf)
    scores = [None] * LQ
    for t in range(LQ):
        sc = jnp.sum(q_emb_ref[:, t, :] * attn_q, axis=1, keepdims=True)
        sc = sc - (1.0 - qm_ref[:, t:t + 1]) * 1e8
        scores[t] = sc
        mx = jnp.maximum(mx, sc)
    ssum = jnp.zeros((B, 1), jnp.float32)
    qv = jnp.zeros((B, H), jnp.float32)
    for t in range(LQ):
        e = jnp.exp(scores[t] - mx)
        ssum = ssum + e
        qv = qv + e * q_emb_ref[:, t, :]
    q_vec_ref[:, 0, :] = qv / ssum

    xr = xr_ref[...][:NUM_REL * LR]
    rproj_f = (xr[:, :4 * HL] + b_f).reshape(NUM_REL, LR, 4 * HL)
    rproj_b = (xr[:, 4 * HL:] + b_b).reshape(NUM_REL, LR, 4 * HL)
    h = jnp.zeros((NUM_REL, HL), jnp.float32)
    c = jnp.zeros((NUM_REL, HL), jnp.float32)
    routs_f = [None] * LR
    for t in range(LR):
        m_col = rm_ref[:, t:t + 1]
        h, c, o = lstm_step(rproj_f[:, t, :], m_col, h, c, whh_f_ref[...])
        routs_f[t] = o
    h = jnp.zeros((NUM_REL, HL), jnp.float32)
    c = jnp.zeros((NUM_REL, HL), jnp.float32)
    for s in range(LR):
        t = LR - 1 - s
        m_col = rm_ref[:, t:t + 1]
        h, c, o = lstm_step(rproj_b[:, t, :], m_col, h, c, whh_b_ref[...])
        relh_ref[t] = jnp.concatenate([routs_f[t], o], axis=1)

    attn_r = attn_r_ref[...]
    mx = jnp.full((NUM_REL, 1), -jnp.inf)
    rscores = [None] * LR
    for t in range(LR):
        sc = jnp.sum(relh_ref[t] * attn_r, axis=1, keepdims=True)
        sc = sc - (1.0 - rm_ref[:, t:t + 1]) * 1e8
        rscores[t] = sc
        mx = jnp.maximum(mx, sc)
    ssum = jnp.zeros((NUM_REL, 1), jnp.float32)
    rv = jnp.zeros((NUM_REL, H), jnp.float32)
    for t in range(LR):
        e = jnp.exp(rscores[t] - mx)
        ssum = ssum + e
        rv = rv + e * relh_ref[t]
    rel_enc_ref[...] = rv / ssum


def _main_kernel(q_emb_ref, qids_ref, rel_ref, rid_ref, eid_ref, qe_ref,
                 ent_ref,
                 ent_lin_b_ref,
                 comb_qrel_W_ref, comb_qrel_b_ref,
                 comb_q_W_ref, comb_q_b_ref,
                 kg_prop_W_ref, kg_prop_b_ref,
                 kg_gate_W_ref, kg_gate_b_ref,
                 q_vec_ref, out_ref):
    rel = rel_ref[...]
    q_emb = q_emb_ref[0]

    qrel = _dot_t(q_emb, rel)
    mq = jnp.max(qrel, axis=1, keepdims=True)
    E = jnp.exp(qrel - mq)
    qrelT = _dot_t(rel, q_emb)
    mqT = jnp.max(qrelT, axis=0, keepdims=True)
    ET = jnp.exp(qrelT - mqT)

    mask_row = (qids_ref[0] != 1).astype(jnp.float32)
    qm = qrelT - (1.0 - mask_row) * 1e20
    qm = qm - jnp.max(qm, axis=1, keepdims=True)
    SqT = jnp.exp(qm)
    SqT = SqT / jnp.sum(SqT, axis=1, keepdims=True)
    Rq = jnp.dot(SqT, q_emb, preferred_element_type=jnp.float32)
    s300 = jnp.sum(rel * Rq, axis=1, keepdims=True)

    Wqr1 = comb_qrel_W_ref[...][:, :H]
    Wqr2 = comb_qrel_W_ref[...][:, H:]
    Aq = _dot_t(q_emb, Wqr1) + comb_qrel_b_ref[...]

    el = _lrelu(ent_ref[...][:, :H] + ent_lin_b_ref[...])

    qe_col = qe_ref[0]

    iota_rel = jax.lax.broadcasted_iota(jnp.int32, (1, 1, NUM_REL), 2)
    iota_ent = jax.lax.broadcasted_iota(jnp.int32, (1, 1, C), 2)

    ent_new_blks = []
    mg_blks = []
    rel_agg_blks = []
    z_blks = []
    for cb in range(C // CB):
        c0 = cb * CB
        rid3 = rid_ref[0, c0:c0 + CB, :][:, :, None]
        eid3 = eid_ref[0, c0:c0 + CB, :][:, :, None]
        oh_rel = (rid3 == iota_rel).astype(jnp.float32)
        oh_ent = (eid3 == iota_ent).astype(jnp.float32)
        cnt = jnp.sum(oh_rel, axis=1)

        oh_rel2 = oh_rel.reshape(CB * N, NUM_REL)
        oh_ent2 = oh_ent.reshape(CB * N, C)
        s_n = jnp.dot(oh_rel2, s300, preferred_element_type=jnp.float32)
        seed = jnp.dot(oh_ent2, qe_col, preferred_element_type=jnp.float32)
        sv = (s_n * seed).reshape(CB, N, 1)
        g = jnp.exp(sv - jnp.max(sv, axis=1, keepdims=True))
        z_blks.append(jnp.sum(g, axis=1))
        cntg = jnp.sum(oh_rel * g, axis=1)
        mg_blks.append(jnp.sum(oh_ent * g, axis=1))
        rel_agg_blks.append(jnp.dot(cntg, rel, preferred_element_type=jnp.float32))

        den = jnp.dot(cnt, ET, preferred_element_type=jnp.float32)
        w_all = (cnt[None, :, :] * E[:, None, :]).reshape(LQ * CB, NUM_REL)
        nq_all = jnp.dot(w_all, rel, preferred_element_type=jnp.float32)
        rq_all = nq_all.reshape(LQ, CB, H) / den.T.reshape(LQ, CB, 1)
        act = jnp.tanh(_dot_t(rq_all.reshape(LQ * CB, H), Wqr2).reshape(LQ, CB, H)
                       + Aq.reshape(LQ, 1, H))
        qn = jnp.max(act, axis=0)
        cqW = comb_q_W_ref[...]
        ent_new_blks.append(_lrelu(_dot_t(el[c0:c0 + CB], cqW[:, :H])
                                   + _dot_t(qn, cqW[:, H:])
                                   + comb_q_b_ref[...]))

    ent_new = jnp.concatenate(ent_new_blks, axis=0)
    mg = jnp.concatenate(mg_blks, axis=0)
    rel_agg = jnp.concatenate(rel_agg_blks, axis=0)
    zz = jnp.concatenate(z_blks, axis=0)

    ent_agg = jnp.dot(mg, ent_new, preferred_element_type=jnp.float32)
    pW = kg_prop_W_ref[...]
    agg = (_dot_t(rel_agg, pW[:, :H]) + _dot_t(ent_agg, pW[:, H:])) / zz \
        + kg_prop_b_ref[...]
    gW = kg_gate_W_ref[...]
    gate = jax.nn.sigmoid(_dot_t(agg, gW[:, :H]) + _dot_t(ent_new, gW[:, H:])
                          + kg_gate_b_ref[...])
    ent2 = gate * _lrelu(agg) + (1.0 - gate) * ent_new
    out_ref[0] = _dot_t(q_vec_ref[0], ent2)


@jax.jit
def kernel(questions, candidate_entities, entity_link_ents, entity_link_rels,
           rel_word_ids, query_entities, entity_table, word_table,
           ent_lin_W, ent_lin_b,
           lstm_Wih_f, lstm_Whh_f, lstm_bih_f, lstm_bhh_f,
           lstm_Wih_b, lstm_Whh_b, lstm_bih_b, lstm_bhh_b,
           attn_r_w, attn_q_w, comb_qrel_W, comb_qrel_b,
           comb_q_W, comb_q_b, kg_prop_W, kg_prop_b, kg_gate_W, kg_gate_b):
    f32 = jnp.float32

    nw = word_table.shape[0]
    ne = entity_table.shape[0]
    wcat = jnp.concatenate([lstm_Wih_f.T, lstm_Wih_b.T], axis=1)
    ew_pad = jnp.zeros((EPROJ, ENT_DIM), f32).at[:H].set(ent_lin_W)

    wblk = 2000
    word_proj = pl.pallas_call(
        _proj_words_kernel,
        grid=(nw // wblk,),
        in_specs=[pl.BlockSpec((wblk, WORD_DIM), lambda i: (i, 0)),
                  pl.BlockSpec((WORD_DIM, WPROJ), lambda i: (0, 0))],
        out_specs=pl.BlockSpec((wblk, WPROJ), lambda i: (i, 0)),
        out_shape=jax.ShapeDtypeStruct((nw, WPROJ), f32),
    )(word_table, wcat)

    eblk = 2048
    ent_proj = pl.pallas_call(
        _proj_ents_kernel,
        grid=(pl.cdiv(ne, eblk),),
        in_specs=[pl.BlockSpec((eblk, ENT_DIM), lambda i: (i, 0)),
                  pl.BlockSpec((EPROJ, ENT_DIM), lambda i: (0, 0))],
        out_specs=pl.BlockSpec((eblk, EPROJ), lambda i: (i, 0)),
        out_shape=jax.ShapeDtypeStruct((ne, EPROJ), f32),
    )(entity_table, ew_pad)

    qids = questions.astype(jnp.int32)
    rids = rel_word_ids.astype(jnp.int32)
    qidx = qids.reshape(XQ_ROWS)
    ridx = jnp.concatenate(
        [rids.reshape(NUM_REL * LR),
         jnp.zeros((XR_ROWS - NUM_REL * LR,), jnp.int32)])
    eidx = candidate_entities.astype(jnp.int32).reshape(ENT_ROWS)
    xq, xr, ent_rows = _gather_sc(word_proj, ent_proj, qidx, ridx, eidx)

    qm = (qids != 1).astype(f32)
    rm = (rids != 1).astype(f32)

    b_f = (lstm_bih_f + lstm_bhh_f).reshape(1, 4 * HL)
    b_b = (lstm_bih_b + lstm_bhh_b).reshape(1, 4 * HL)

    q_emb, q_vec, rel_encoded = pl.pallas_call(
        _encode_kernel,
        out_shape=[
            jax.ShapeDtypeStruct((B, LQ, H), f32),
            jax.ShapeDtypeStruct((B, 1, H), f32),
            jax.ShapeDtypeStruct((NUM_REL, H), f32),
        ],
        scratch_shapes=[pltpu.VMEM((LR, NUM_REL, H), f32)],
    )(xq, xr, qm, rm,
      lstm_Whh_f, b_f,
      lstm_Whh_b, b_b,
      attn_r_w.reshape(1, H), attn_q_w.reshape(1, H))

    qids3 = questions.astype(jnp.int32).reshape(B, 1, LQ)
    qe3 = query_entities.astype(f32).reshape(B, C, 1)
    rid = entity_link_rels.astype(jnp.int32)
    eid = entity_link_ents.astype(jnp.int32)

    full = lambda shape: pl.BlockSpec(shape, lambda b: tuple(0 for _ in shape))
    row = lambda shape: pl.BlockSpec(shape, lambda b: (b,) + tuple(0 for _ in shape[1:]))

    out = pl.pallas_call(
        _main_kernel,
        grid=(B,),
        in_specs=[
            row((1, LQ, H)),
            row((1, 1, LQ)),
            full((NUM_REL, H)),
            row((1, C, N)),
            row((1, C, N)),
            row((1, C, 1)),
            pl.BlockSpec((C, EPROJ), lambda b: (b, 0)),
            full((1, H)),
            full((H, 2 * H)),
            full((1, H)),
            full((H, 2 * H)),
            full((1, H)),
            full((H, 2 * H)),
            full((1, H)),
            full((H, 2 * H)),
            full((1, H)),
            row((1, 1, H)),
        ],
        out_specs=pl.BlockSpec((1, 1, C), lambda b: (b, 0, 0)),
        out_shape=jax.ShapeDtypeStruct((B, 1, C), f32),
        compiler_params=pltpu.CompilerParams(
            dimension_semantics=("arbitrary",),
        ),
    )(q_emb, qids3, rel_encoded, rid, eid, qe3, ent_rows,
      ent_lin_b.reshape(1, H),
      comb_qrel_W, comb_qrel_b.reshape(1, H),
      comb_q_W, comb_q_b.reshape(1, H),
      kg_prop_W, kg_prop_b.reshape(1, H),
      kg_gate_W, kg_gate_b.reshape(1, H),
      q_vec)
    return out.reshape(B, C)
